# baseline (speedup 1.0000x reference)
import functools

import jax
import jax.numpy as jnp
from jax import lax
from jax.experimental import pallas as pl
from jax.experimental.pallas import tpu as pltpu
from jax.experimental.pallas import tpu_sc as plsc

NC = 2
NS = 16
K = 128
BN = 1000


def _sc_mesh():
  return plsc.VectorSubcoreMesh(core_axis_name="c", subcore_axis_name="s")


_DEG_CH = 1024


def _deg_body(npad, epad, dst_hbm, out_hbm, didx, hist):
  c = lax.axis_index("c")
  s = lax.axis_index("s")
  rpt = npad // NS
  half = epad // NC
  zero = jnp.zeros((16,), jnp.float32)
  ones16 = jnp.ones((16,), jnp.float32)
  lane = jnp.arange(16, dtype=jnp.int32)
  base = s * rpt

  def zrow(i, _):
    hist[i, :] = zero
    return 0
  lax.fori_loop(0, rpt, zrow, 0)

  def chunk(k, _):
    pltpu.sync_copy(dst_hbm.at[pl.ds(c * half + k * _DEG_CH, _DEG_CH)], didx)

    def inner(i, _):
      d = didx[pl.ds(i * 16, 16)]
      local = d - base
      m = (local >= 0) & (local < rpt)
      local = jnp.where(m, local, 0)
      plsc.addupdate_scatter(hist, [local, lane], ones16, mask=m)
      return 0
    lax.fori_loop(0, _DEG_CH // 16, inner, 0)
    return 0
  lax.fori_loop(0, half // _DEG_CH, chunk, 0)
  pltpu.sync_copy(hist, out_hbm.at[c, pl.ds(base, rpt)])


def _sc_degree(dst_p, npad, epad):
  kern = pl.kernel(
      functools.partial(_deg_body, npad, epad),
      out_type=jax.ShapeDtypeStruct((NC, npad, 16), jnp.float32),
      mesh=_sc_mesh(),
      scratch_types=[
          pltpu.VMEM((_DEG_CH,), jnp.int32),
          pltpu.VMEM((npad // NS, 16), jnp.float32),
      ],
      compiler_params=pltpu.CompilerParams(needs_layout_passes=False),
  )
  return kern(dst_p)


def _dred_body(h_ref, out_ref):
  deg = jnp.sum(h_ref[...], axis=(0, 2)) + 1.0
  dinv = lax.rsqrt(deg).reshape(-1, 1)
  out_ref[...] = jnp.broadcast_to(dinv, (dinv.shape[0], 8))


def _tc_deg_reduce(degp, npad):
  bn = 1264
  grid_spec = pl.GridSpec(
      grid=(npad // bn,),
      in_specs=[pl.BlockSpec((NC, bn, 16), lambda i: (0, i, 0))],
      out_specs=pl.BlockSpec((bn, 8), lambda i: (i, 0)),
  )
  return pl.pallas_call(
      _dred_body, grid_spec=grid_spec,
      out_shape=jax.ShapeDtypeStruct((npad, 8), jnp.float32))(degp)


def _agg_body(npad, nchunks, src_hbm, dst_hbm, t0, t1, t2, t3,
              o0, o1, o2, o3, sidxa, sidxb, didxa, didxb, rows0, rows1,
              zbuf, aggsp, g0s, g1s, sas, sbs, ias, ibs):
  c = lax.axis_index("c")
  s = lax.axis_index("s")
  wid = c * NS + s
  rows_per_tile = npad // NS
  zero = jnp.zeros((16,), jnp.float32)
  ebase = wid * nchunks * K
  last = nchunks - 1

  def zrow(i, _):
    for j in range(8):
      zbuf[i, pl.ds(j * 16, 16)] = zero
    return 0
  lax.fori_loop(0, K, zrow, 0)

  for tbl, out in ((t0, o0), (t1, o1), (t2, o2), (t3, o3)):
    for r in range(rows_per_tile // K):
      pltpu.sync_copy(zbuf, aggsp.at[pl.ds(s * rows_per_tile + r * K, K)])
    rem = rows_per_tile % K
    if rem:
      pltpu.sync_copy(
          zbuf.at[pl.ds(0, rem)],
          aggsp.at[pl.ds(s * rows_per_tile + (rows_per_tile // K) * K, rem)])
    plsc.subcore_barrier()

    pltpu.sync_copy(src_hbm.at[pl.ds(ebase, K)], sidxa)
    pltpu.sync_copy(src_hbm.at[pl.ds(ebase + K, K)], sidxb)
    pltpu.async_copy(tbl.at[sidxa], rows0, g0s)
    pltpu.async_copy(tbl.at[sidxb], rows1, g1s)
    pltpu.async_copy(dst_hbm.at[pl.ds(ebase, K)], didxa, ias)
    pltpu.async_copy(dst_hbm.at[pl.ds(ebase + K, K)], didxb, ibs)

    def body(p, _):
      k2 = jnp.minimum(2 * p + 2, last)
      k3 = jnp.minimum(2 * p + 3, last)
      e2 = ebase + k2 * K
      e3 = ebase + k3 * K
      pltpu.make_async_copy(tbl.at[sidxa], rows0, g0s).wait()
      pltpu.async_copy(src_hbm.at[pl.ds(e2, K)], sidxa, sas)
      pltpu.make_async_copy(dst_hbm.at[pl.ds(e2, K)], didxa, ias).wait()
      pltpu.sync_copy(rows0, aggsp.at[didxa], add=True)
      pltpu.make_async_copy(src_hbm.at[pl.ds(e2, K)], sidxa, sas).wait()
      pltpu.async_copy(tbl.at[sidxa], rows0, g0s)
      pltpu.async_copy(dst_hbm.at[pl.ds(e2, K)], didxa, ias)
      pltpu.make_async_copy(tbl.at[sidxb], rows1, g1s).wait()
      pltpu.async_copy(src_hbm.at[pl.ds(e3, K)], sidxb, sbs)
      pltpu.make_async_copy(dst_hbm.at[pl.ds(e3, K)], didxb, ibs).wait()
      pltpu.sync_copy(rows1, aggsp.at[didxb], add=True)
      pltpu.make_async_copy(src_hbm.at[pl.ds(e3, K)], sidxb, sbs).wait()
      pltpu.async_copy(tbl.at[sidxb], rows1, g1s)
      pltpu.async_copy(dst_hbm.at[pl.ds(e3, K)], didxb, ibs)
      return 0
    lax.fori_loop(0, nchunks // 2, body, 0)
    pltpu.make_async_copy(tbl.at[sidxa], rows0, g0s).wait()
    pltpu.make_async_copy(tbl.at[sidxb], rows1, g1s).wait()
    pltpu.make_async_copy(dst_hbm.at[pl.ds(ebase, K)], didxa, ias).wait()
    pltpu.make_async_copy(dst_hbm.at[pl.ds(ebase, K)], didxb, ibs).wait()
    plsc.subcore_barrier()
    pltpu.sync_copy(aggsp.at[pl.ds(s * rows_per_tile, rows_per_tile)],
                    out.at[c, pl.ds(s * rows_per_tile, rows_per_tile)])
    plsc.subcore_barrier()


def _sc_edge_agg(src_r, dst_r, tables, npad, nchunks):
  pshape = jax.ShapeDtypeStruct((NC, npad, 128), jnp.float32)
  kern = pl.kernel(
      functools.partial(_agg_body, npad, nchunks),
      out_type=[pshape] * 4,
      mesh=_sc_mesh(),
      scratch_types=[
          pltpu.VMEM((K,), jnp.int32),
          pltpu.VMEM((K,), jnp.int32),
          pltpu.VMEM((K,), jnp.int32),
          pltpu.VMEM((K,), jnp.int32),
          pltpu.VMEM((K, 128), jnp.float32),
          pltpu.VMEM((K, 128), jnp.float32),
          pltpu.VMEM((K, 128), jnp.float32),
          pltpu.VMEM_SHARED((npad, 128), jnp.float32),
          pltpu.SemaphoreType.DMA,
          pltpu.SemaphoreType.DMA,
          pltpu.SemaphoreType.DMA,
          pltpu.SemaphoreType.DMA,
          pltpu.SemaphoreType.DMA,
          pltpu.SemaphoreType.DMA,
      ],
  )
  return kern(src_r, dst_r, *tables)


def _mm1_body(x_ref, dv_ref, w_ref, *outs):
  dinv = dv_ref[:, 0:1]
  for c in range(4):
    for l in range(2):
      g = jnp.dot(x_ref[2 * c + l], w_ref[...],
                  preferred_element_type=jnp.float32)
      outs[c][:, l * 64:(l + 1) * 64] = g * dinv


def _tc_mm1(xr, dinv8, W_g1, n, npad):
  nblk = n // BN
  out = jax.ShapeDtypeStruct((npad, 128), jnp.float32)
  grid_spec = pl.GridSpec(
      grid=(nblk,),
      in_specs=[
          pl.BlockSpec((8, BN, 128), lambda i: (0, i, 0)),
          pl.BlockSpec((BN, 8), lambda i: (i, 0)),
          pl.BlockSpec((128, 64), lambda i: (0, 0)),
      ],
      out_specs=[pl.BlockSpec((BN, 128), lambda i: (i, 0))] * 4,
  )
  return pl.pallas_call(_mm1_body, grid_spec=grid_spec,
                        out_shape=[out] * 4)(xr, dinv8, W_g1)


def _mm2_body(dv_ref, p0, p1, p2, p3, t0, t1, t2, t3, b1_ref, w_ref,
              o0, o1, o2, o3):
  dinv = dv_ref[:, 0:1]
  b1rep = jnp.concatenate([b1_ref[...], b1_ref[...]], axis=1)
  ps = (p0, p1, p2, p3)
  ts = (t0, t1, t2, t3)
  outs = (o0, o1, o2, o3)
  for c in range(4):
    h1 = jax.nn.relu((ps[c][0] + ps[c][1] + ts[c][...]) * dinv + b1rep)
    for l in range(2):
      g = jnp.dot(h1[:, l * 64:(l + 1) * 64], w_ref[...],
                  preferred_element_type=jnp.float32)
      outs[c][:, l * 64:(l + 1) * 64] = g * dinv


def _tc_mm2(dinv8, parts, tabs, b_g1, W_g2, n, npad):
  nblk = n // BN
  out = jax.ShapeDtypeStruct((npad, 128), jnp.float32)
  grid_spec = pl.GridSpec(
      grid=(nblk,),
      in_specs=(
          [pl.BlockSpec((BN, 8), lambda i: (i, 0))]
          + [pl.BlockSpec((NC, BN, 128), lambda i: (0, i, 0))] * 4
          + [pl.BlockSpec((BN, 128), lambda i: (i, 0))] * 4
          + [pl.BlockSpec((1, 64), lambda i: (0, 0)),
             pl.BlockSpec((64, 64), lambda i: (0, 0))]
      ),
      out_specs=[pl.BlockSpec((BN, 128), lambda i: (i, 0))] * 4,
  )
  return pl.pallas_call(_mm2_body, grid_spec=grid_spec,
                        out_shape=[out] * 4)(
                            dinv8, *parts, *tabs, b_g1.reshape(1, 64), W_g2)


def _lstm_body(dv_ref, q0, q1, q2, q3, u0, u1, u2, u3, b2_ref,
               wih_ref, whh_ref, bsum_ref, wlin_ref, blin_ref, out_ref):
  dinv = dv_ref[:, 0:1]
  b2rep = jnp.concatenate([b2_ref[...], b2_ref[...]], axis=1)
  qs = (q0, q1, q2, q3)
  us = (u0, u1, u2, u3)
  xs = []
  for c in range(4):
    h2 = (qs[c][0] + qs[c][1] + us[c][...]) * dinv + b2rep
    xs.append(h2[:, :64])
    xs.append(h2[:, 64:])
  h = jnp.zeros((2 * BN, 64), jnp.float32)
  cc = jnp.zeros((2 * BN, 64), jnp.float32)
  wih = wih_ref[...]
  whh = whh_ref[...]
  bsum = bsum_ref[...]
  for t in range(4):
    xt = jnp.concatenate([xs[t], xs[4 + t]], axis=0)
    gates = (jnp.dot(xt, wih, preferred_element_type=jnp.float32)
             + jnp.dot(h, whh, preferred_element_type=jnp.float32) + bsum)
    i = jax.nn.sigmoid(gates[:, :64])
    f = jax.nn.sigmoid(gates[:, 64:128])
    g = jnp.tanh(gates[:, 128:192])
    o = jax.nn.sigmoid(gates[:, 192:256])
    cc = f * cc + i * g
    h = o * jnp.tanh(cc)
  res = jnp.dot(h, wlin_ref[...], preferred_element_type=jnp.float32)
  res = res + blin_ref[...]
  out_ref[...] = res.reshape(2, BN, 128)


def _tc_lstm(dinv8, parts, tabs, b_g2, W_ihT, W_hhT, bsum, W_linp, blinp, n):
  nblk = n // BN
  out = jax.ShapeDtypeStruct((2, n, 128), jnp.float32)
  grid_spec = pl.GridSpec(
      grid=(nblk,),
      in_specs=(
          [pl.BlockSpec((BN, 8), lambda i: (i, 0))]
          + [pl.BlockSpec((NC, BN, 128), lambda i: (0, i, 0))] * 4
          + [pl.BlockSpec((BN, 128), lambda i: (i, 0))] * 4
          + [pl.BlockSpec((1, 64), lambda i: (0, 0)),
             pl.BlockSpec((64, 256), lambda i: (0, 0)),
             pl.BlockSpec((64, 256), lambda i: (0, 0)),
             pl.BlockSpec((1, 256), lambda i: (0, 0)),
             pl.BlockSpec((64, 128), lambda i: (0, 0)),
             pl.BlockSpec((1, 128), lambda i: (0, 0))]
      ),
      out_specs=[pl.BlockSpec((2, BN, 128), lambda i: (0, i, 0))],
  )
  return pl.pallas_call(_lstm_body, grid_spec=grid_spec,
                        out_shape=[out])(
                            dinv8, *parts, *tabs, b_g2.reshape(1, 64),
                            W_ihT, W_hhT, bsum, W_linp, blinp)[0]


def kernel(x, W_g1, b_g1, W_g2, b_g2, W_ih, W_hh, b_ih, b_hh, W_lin, b_lin,
           edge_index):
  B, T, N, CIN = x.shape
  E = edge_index.shape[1]
  npad = ((N + 1 + 127) // 128) * 128
  lanes = NC * NS
  epad = ((E + K * lanes - 1) // (K * lanes)) * (K * lanes)
  nchunks = epad // (K * lanes)

  pad = jnp.full((epad - E,), N, dtype=jnp.int32)
  pad_dst = N + (jnp.arange(epad - E, dtype=jnp.int32) % (npad - N))
  order = jnp.argsort(edge_index[0])
  src_p = jnp.concatenate([edge_index[0][order], pad])
  dst_p = jnp.concatenate([edge_index[1][order], pad_dst])
  xr = x.reshape(B * T, N, CIN)

  degp = _sc_degree(dst_p, npad, epad)
  dinv8 = _tc_deg_reduce(degp, npad)
  tabs1 = _tc_mm1(xr, dinv8, W_g1, N, npad)
  parts1 = _sc_edge_agg(src_p, dst_p, tabs1, npad, nchunks)
  tabs2 = _tc_mm2(dinv8, parts1, tabs1, b_g1, W_g2, N, npad)
  parts2 = _sc_edge_agg(src_p, dst_p, tabs2, npad, nchunks)

  W_linp = jnp.pad(W_lin, ((0, 0), (0, 127)))
  blinp = jnp.pad(b_lin.reshape(1, 1), ((0, 0), (0, 127)))
  bsum = (b_ih + b_hh).reshape(1, 256)
  res = _tc_lstm(dinv8, parts2, tabs2, b_g2, W_ih.T, W_hh.T, bsum,
                 W_linp, blinp, N)
  return res[:, :, :1]


# speedup vs baseline: 1.0586x; 1.0586x over previous
import functools

import jax
import jax.numpy as jnp
from jax import lax
from jax.experimental import pallas as pl
from jax.experimental.pallas import tpu as pltpu
from jax.experimental.pallas import tpu_sc as plsc

NC = 2
NS = 16
K = 128
BN = 1000


def _sc_mesh():
  return plsc.VectorSubcoreMesh(core_axis_name="c", subcore_axis_name="s")


_DEG_CH = 1024


def _deg_body(npad, epad, dst_hbm, out_hbm, didx, hist):
  c = lax.axis_index("c")
  s = lax.axis_index("s")
  rpt = npad // NS
  half = epad // NC
  zero = jnp.zeros((16,), jnp.float32)
  ones16 = jnp.ones((16,), jnp.float32)
  lane = jnp.arange(16, dtype=jnp.int32)
  base = s * rpt

  def zrow(i, _):
    hist[i, :] = zero
    return 0
  lax.fori_loop(0, rpt, zrow, 0)

  def chunk(k, _):
    pltpu.sync_copy(dst_hbm.at[pl.ds(c * half + k * _DEG_CH, _DEG_CH)], didx)

    def inner(i, _):
      d = didx[pl.ds(i * 16, 16)]
      local = d - base
      m = (local >= 0) & (local < rpt)
      local = jnp.where(m, local, 0)
      plsc.addupdate_scatter(hist, [local, lane], ones16, mask=m)
      return 0
    lax.fori_loop(0, _DEG_CH // 16, inner, 0)
    return 0
  lax.fori_loop(0, half // _DEG_CH, chunk, 0)
  pltpu.sync_copy(hist, out_hbm.at[c, pl.ds(base, rpt)])


def _sc_degree(dst_p, npad, epad):
  kern = pl.kernel(
      functools.partial(_deg_body, npad, epad),
      out_type=jax.ShapeDtypeStruct((NC, npad, 16), jnp.float32),
      mesh=_sc_mesh(),
      scratch_types=[
          pltpu.VMEM((_DEG_CH,), jnp.int32),
          pltpu.VMEM((npad // NS, 16), jnp.float32),
      ],
      compiler_params=pltpu.CompilerParams(needs_layout_passes=False),
  )
  return kern(dst_p)


def _dred_body(h_ref, out_ref):
  deg = jnp.sum(h_ref[...], axis=(0, 2)) + 1.0
  dinv = lax.rsqrt(deg).reshape(-1, 1)
  out_ref[...] = jnp.broadcast_to(dinv, (dinv.shape[0], 8))


def _tc_deg_reduce(degp, npad):
  bn = 1264
  grid_spec = pl.GridSpec(
      grid=(npad // bn,),
      in_specs=[pl.BlockSpec((NC, bn, 16), lambda i: (0, i, 0))],
      out_specs=pl.BlockSpec((bn, 8), lambda i: (i, 0)),
  )
  return pl.pallas_call(
      _dred_body, grid_spec=grid_spec,
      out_shape=jax.ShapeDtypeStruct((npad, 8), jnp.float32))(degp)


def _agg_body(npad, nchunks, src_hbm, dst_hbm, t0, t1, t2, t3,
              o0, o1, o2, o3, sidx0, sidx1, didx0, didx1, rows0, rows1,
              zbuf, aggsp, sem0, sem1):
  c = lax.axis_index("c")
  s = lax.axis_index("s")
  wid = c * NS + s
  rows_per_tile = npad // NS
  zero = jnp.zeros((16,), jnp.float32)
  ebase = wid * nchunks * K
  last = nchunks - 1

  def zrow(i, _):
    for j in range(8):
      zbuf[i, pl.ds(j * 16, 16)] = zero
    return 0
  lax.fori_loop(0, K, zrow, 0)

  for tbl, out in ((t0, o0), (t1, o1), (t2, o2), (t3, o3)):
    for r in range(rows_per_tile // K):
      pltpu.sync_copy(zbuf, aggsp.at[pl.ds(s * rows_per_tile + r * K, K)])
    rem = rows_per_tile % K
    if rem:
      pltpu.sync_copy(
          zbuf.at[pl.ds(0, rem)],
          aggsp.at[pl.ds(s * rows_per_tile + (rows_per_tile // K) * K, rem)])
    plsc.subcore_barrier()

    pltpu.sync_copy(src_hbm.at[pl.ds(ebase, K)], sidx0)
    pltpu.async_copy(tbl.at[sidx0], rows0, sem0)

    def body(p, _):
      k0 = 2 * p
      k1 = 2 * p + 1
      k2 = jnp.minimum(k1 + 1, last)
      pltpu.sync_copy(src_hbm.at[pl.ds(ebase + k1 * K, K)], sidx1)
      g1 = pltpu.async_copy(tbl.at[sidx1], rows1, sem1)
      pltpu.make_async_copy(tbl.at[sidx0], rows0, sem0).wait()
      pltpu.sync_copy(dst_hbm.at[pl.ds(ebase + k0 * K, K)], didx0)
      pltpu.sync_copy(src_hbm.at[pl.ds(ebase + k2 * K, K)], sidx0)
      pltpu.async_copy(tbl.at[sidx0], rows0, sem0)
      pltpu.sync_copy(rows0, aggsp.at[didx0], add=True)
      g1.wait()
      pltpu.sync_copy(dst_hbm.at[pl.ds(ebase + k1 * K, K)], didx1)
      pltpu.sync_copy(rows1, aggsp.at[didx1], add=True)
      return 0
    lax.fori_loop(0, nchunks // 2, body, 0)
    pltpu.make_async_copy(tbl.at[sidx0], rows0, sem0).wait()
    plsc.subcore_barrier()
    pltpu.sync_copy(aggsp.at[pl.ds(s * rows_per_tile, rows_per_tile)],
                    out.at[c, pl.ds(s * rows_per_tile, rows_per_tile)])
    plsc.subcore_barrier()


def _sc_edge_agg(src_r, dst_r, tables, npad, nchunks):
  pshape = jax.ShapeDtypeStruct((NC, npad, 128), jnp.float32)
  kern = pl.kernel(
      functools.partial(_agg_body, npad, nchunks),
      out_type=[pshape] * 4,
      mesh=_sc_mesh(),
      scratch_types=[
          pltpu.VMEM((K,), jnp.int32),
          pltpu.VMEM((K,), jnp.int32),
          pltpu.VMEM((K,), jnp.int32),
          pltpu.VMEM((K,), jnp.int32),
          pltpu.VMEM((K, 128), jnp.float32),
          pltpu.VMEM((K, 128), jnp.float32),
          pltpu.VMEM((K, 128), jnp.float32),
          pltpu.VMEM_SHARED((npad, 128), jnp.float32),
          pltpu.SemaphoreType.DMA,
          pltpu.SemaphoreType.DMA,
      ],
  )
  return kern(src_r, dst_r, *tables)


def _mm1_body(x_ref, dv_ref, w_ref, *outs):
  dinv = dv_ref[:, 0:1]
  for c in range(4):
    for l in range(2):
      g = jnp.dot(x_ref[2 * c + l], w_ref[...],
                  preferred_element_type=jnp.float32)
      outs[c][:, l * 64:(l + 1) * 64] = g * dinv


def _tc_mm1(xr, dinv8, W_g1, n, npad):
  nblk = n // BN
  out = jax.ShapeDtypeStruct((npad, 128), jnp.float32)
  grid_spec = pl.GridSpec(
      grid=(nblk,),
      in_specs=[
          pl.BlockSpec((8, BN, 128), lambda i: (0, i, 0)),
          pl.BlockSpec((BN, 8), lambda i: (i, 0)),
          pl.BlockSpec((128, 64), lambda i: (0, 0)),
      ],
      out_specs=[pl.BlockSpec((BN, 128), lambda i: (i, 0))] * 4,
  )
  return pl.pallas_call(_mm1_body, grid_spec=grid_spec,
                        out_shape=[out] * 4)(xr, dinv8, W_g1)


def _mm2_body(dv_ref, p0, p1, p2, p3, t0, t1, t2, t3, b1_ref, w_ref,
              o0, o1, o2, o3):
  dinv = dv_ref[:, 0:1]
  b1rep = jnp.concatenate([b1_ref[...], b1_ref[...]], axis=1)
  ps = (p0, p1, p2, p3)
  ts = (t0, t1, t2, t3)
  outs = (o0, o1, o2, o3)
  for c in range(4):
    h1 = jax.nn.relu((ps[c][0] + ps[c][1] + ts[c][...]) * dinv + b1rep)
    for l in range(2):
      g = jnp.dot(h1[:, l * 64:(l + 1) * 64], w_ref[...],
                  preferred_element_type=jnp.float32)
      outs[c][:, l * 64:(l + 1) * 64] = g * dinv


def _tc_mm2(dinv8, parts, tabs, b_g1, W_g2, n, npad):
  nblk = n // BN
  out = jax.ShapeDtypeStruct((npad, 128), jnp.float32)
  grid_spec = pl.GridSpec(
      grid=(nblk,),
      in_specs=(
          [pl.BlockSpec((BN, 8), lambda i: (i, 0))]
          + [pl.BlockSpec((NC, BN, 128), lambda i: (0, i, 0))] * 4
          + [pl.BlockSpec((BN, 128), lambda i: (i, 0))] * 4
          + [pl.BlockSpec((1, 64), lambda i: (0, 0)),
             pl.BlockSpec((64, 64), lambda i: (0, 0))]
      ),
      out_specs=[pl.BlockSpec((BN, 128), lambda i: (i, 0))] * 4,
  )
  return pl.pallas_call(_mm2_body, grid_spec=grid_spec,
                        out_shape=[out] * 4)(
                            dinv8, *parts, *tabs, b_g1.reshape(1, 64), W_g2)


def _lstm_body(dv_ref, q0, q1, q2, q3, u0, u1, u2, u3, b2_ref,
               wih_ref, whh_ref, bsum_ref, wlin_ref, blin_ref, out_ref):
  dinv = dv_ref[:, 0:1]
  b2rep = jnp.concatenate([b2_ref[...], b2_ref[...]], axis=1)
  qs = (q0, q1, q2, q3)
  us = (u0, u1, u2, u3)
  xs = []
  for c in range(4):
    h2 = (qs[c][0] + qs[c][1] + us[c][...]) * dinv + b2rep
    xs.append(h2[:, :64])
    xs.append(h2[:, 64:])
  h = jnp.zeros((2 * BN, 64), jnp.float32)
  cc = jnp.zeros((2 * BN, 64), jnp.float32)
  wih = wih_ref[...]
  whh = whh_ref[...]
  bsum = bsum_ref[...]
  for t in range(4):
    xt = jnp.concatenate([xs[t], xs[4 + t]], axis=0)
    gates = (jnp.dot(xt, wih, preferred_element_type=jnp.float32)
             + jnp.dot(h, whh, preferred_element_type=jnp.float32) + bsum)
    i = jax.nn.sigmoid(gates[:, :64])
    f = jax.nn.sigmoid(gates[:, 64:128])
    g = jnp.tanh(gates[:, 128:192])
    o = jax.nn.sigmoid(gates[:, 192:256])
    cc = f * cc + i * g
    h = o * jnp.tanh(cc)
  res = jnp.dot(h, wlin_ref[...], preferred_element_type=jnp.float32)
  res = res + blin_ref[...]
  out_ref[...] = res.reshape(2, BN, 128)


def _tc_lstm(dinv8, parts, tabs, b_g2, W_ihT, W_hhT, bsum, W_linp, blinp, n):
  nblk = n // BN
  out = jax.ShapeDtypeStruct((2, n, 128), jnp.float32)
  grid_spec = pl.GridSpec(
      grid=(nblk,),
      in_specs=(
          [pl.BlockSpec((BN, 8), lambda i: (i, 0))]
          + [pl.BlockSpec((NC, BN, 128), lambda i: (0, i, 0))] * 4
          + [pl.BlockSpec((BN, 128), lambda i: (i, 0))] * 4
          + [pl.BlockSpec((1, 64), lambda i: (0, 0)),
             pl.BlockSpec((64, 256), lambda i: (0, 0)),
             pl.BlockSpec((64, 256), lambda i: (0, 0)),
             pl.BlockSpec((1, 256), lambda i: (0, 0)),
             pl.BlockSpec((64, 128), lambda i: (0, 0)),
             pl.BlockSpec((1, 128), lambda i: (0, 0))]
      ),
      out_specs=[pl.BlockSpec((2, BN, 128), lambda i: (0, i, 0))],
  )
  return pl.pallas_call(_lstm_body, grid_spec=grid_spec,
                        out_shape=[out])(
                            dinv8, *parts, *tabs, b_g2.reshape(1, 64),
                            W_ihT, W_hhT, bsum, W_linp, blinp)[0]


def kernel(x, W_g1, b_g1, W_g2, b_g2, W_ih, W_hh, b_ih, b_hh, W_lin, b_lin,
           edge_index):
  B, T, N, CIN = x.shape
  E = edge_index.shape[1]
  npad = ((N + 1 + 127) // 128) * 128
  lanes = NC * NS
  epad = ((E + K * lanes - 1) // (K * lanes)) * (K * lanes)
  nchunks = epad // (K * lanes)

  pad = jnp.full((epad - E,), N, dtype=jnp.int32)
  pad_dst = N + (jnp.arange(epad - E, dtype=jnp.int32) % (npad - N))
  src_p = jnp.concatenate([edge_index[0], pad])
  dst_p = jnp.concatenate([edge_index[1], pad_dst])
  xr = x.reshape(B * T, N, CIN)

  degp = _sc_degree(dst_p, npad, epad)
  dinv8 = _tc_deg_reduce(degp, npad)
  tabs1 = _tc_mm1(xr, dinv8, W_g1, N, npad)
  parts1 = _sc_edge_agg(src_p, dst_p, tabs1, npad, nchunks)
  tabs2 = _tc_mm2(dinv8, parts1, tabs1, b_g1, W_g2, N, npad)
  parts2 = _sc_edge_agg(src_p, dst_p, tabs2, npad, nchunks)

  W_linp = jnp.pad(W_lin, ((0, 0), (0, 127)))
  blinp = jnp.pad(b_lin.reshape(1, 1), ((0, 0), (0, 127)))
  bsum = (b_ih + b_hh).reshape(1, 256)
  res = _tc_lstm(dinv8, parts2, tabs2, b_g2, W_ih.T, W_hh.T, bsum,
                 W_linp, blinp, N)
  return res[:, :, :1]
